# Initial kernel scaffold; baseline (speedup 1.0000x reference)
import jax
import jax.numpy as jnp
from jax.experimental import pallas as pl


def kernel(visit_order, pos_embed):
    raise NotImplementedError("write your pallas kernel here")


# speedup vs baseline: 4.1453x; 4.1453x over previous
import functools

import jax
import jax.numpy as jnp
from jax import lax
from jax.experimental import pallas as pl
from jax.experimental.pallas import tpu as pltpu
from jax.experimental.pallas import tpu_sc as plsc

_NC = 2
_NS = 16
_NW = _NC * _NS

_CHUNK = 128
_GATHERS = 8
_BLOCK = _CHUNK * _GATHERS


@functools.lru_cache(maxsize=None)
def _build(B, V, D):
    assert B % (_NW * _BLOCK) == 0
    per_w = B // _NW
    nblk = per_w // _BLOCK

    mesh = plsc.VectorSubcoreMesh(core_axis_name="c", subcore_axis_name="s")

    @functools.partial(
        pl.kernel,
        out_type=jax.ShapeDtypeStruct((B, D), jnp.float32),
        mesh=mesh,
        scratch_types=[
            pltpu.VMEM((_BLOCK,), jnp.int32),
            pltpu.VMEM((_BLOCK, D), jnp.float32),
            pltpu.SemaphoreType.DMA,
        ],
        compiler_params=pltpu.CompilerParams(use_tc_tiling_on_sc=False),
    )
    def emb(idx_hbm, table_hbm, out_hbm, idx_v, rows_v, gsem):
        wid = lax.axis_index("s") * _NC + lax.axis_index("c")
        base = wid * per_w

        def body(blk, carry):
            off = pl.multiple_of(base + blk * _BLOCK, _BLOCK)
            pltpu.sync_copy(idx_hbm.at[pl.ds(off, _BLOCK)], idx_v)
            copies = [
                pltpu.async_copy(
                    table_hbm.at[idx_v.at[pl.ds(j * _CHUNK, _CHUNK)]],
                    rows_v.at[pl.ds(j * _CHUNK, _CHUNK)],
                    gsem,
                )
                for j in range(_GATHERS)
            ]
            for c in copies:
                c.wait()
            pltpu.sync_copy(rows_v, out_hbm.at[pl.ds(off, _BLOCK)])
            return carry

        lax.fori_loop(0, nblk, body, 0)

    return emb


def kernel(visit_order, pos_embed):
    R, S = visit_order.shape
    V, D = pos_embed.shape
    B = R * S
    idx = visit_order.reshape(B).astype(jnp.int32)
    out = _build(B, V, D)(idx, pos_embed)
    return out.reshape(R, S, D)
